# baseline (speedup 1.0000x reference)
import jax
import jax.numpy as jnp
from jax import lax
from jax.experimental import pallas as pl
from jax.experimental.pallas import tpu as pltpu
from jax.experimental.pallas import tpu_sc as plsc

N = 10000
D = 128
E = 320000
NC = 2
NS = 16
NW = NC * NS
K = 128
NCHUNK = 80
NCH_PH = 32
CF = 160
CS = 2 * NCHUNK - CF
FAST_C = 0
EPAD = NW * NCHUNK * K
JUNK = 512
ACC_ROWS = N + JUNK
IOSUB = 10
IOROWS = N // IOSUB


def _sc_body(feat_hbm, src_hbm, dst_hbm, out_hbm, acc, idx_s, idx_d,
             rows0, rows1, sem0, sem1):
    c = lax.axis_index("c")
    s = lax.axis_index("s")

    io_base = pl.multiple_of(s * IOROWS, 8)

    @pl.when(s < IOSUB)
    def _init():
        pltpu.sync_copy(feat_hbm.at[pl.ds(io_base, IOROWS)],
                        acc.at[pl.ds(io_base, IOROWS)])

    plsc.subcore_barrier()

    nph = jnp.where(c == FAST_C, CF // NCH_PH, CS // NCH_PH)
    base = jnp.where(c == FAST_C, s * CF, NS * CF + s * CS)

    def phase(p, pcarry):
        slab = pl.multiple_of(base + p * NCH_PH, 8)
        pltpu.sync_copy(src_hbm.at[pl.ds(slab, NCH_PH)], idx_s)
        pltpu.sync_copy(dst_hbm.at[pl.ds(slab, NCH_PH)], idx_d)

        pltpu.async_copy(feat_hbm.at[idx_s.at[0]], rows0, sem0)

        def body(i, carry):
            j0 = 2 * i
            j1 = j0 + 1
            pltpu.async_copy(feat_hbm.at[idx_s.at[j1]], rows1, sem1)
            pltpu.make_async_copy(feat_hbm.at[idx_s.at[j0]], rows0, sem0).wait()
            pltpu.sync_copy(rows0, acc.at[idx_d.at[j0]], add=True)

            @pl.when(j1 + 1 < NCH_PH)
            def _():
                pltpu.async_copy(feat_hbm.at[idx_s.at[j1 + 1]], rows0, sem0)

            pltpu.make_async_copy(feat_hbm.at[idx_s.at[j1]], rows1, sem1).wait()
            pltpu.sync_copy(rows1, acc.at[idx_d.at[j1]], add=True)
            return carry

        lax.fori_loop(0, NCH_PH // 2, body, 0)
        return pcarry

    lax.fori_loop(0, nph, phase, 0)

    plsc.subcore_barrier()

    @pl.when(s < IOSUB)
    def _writeout():
        pltpu.sync_copy(acc.at[pl.ds(io_base, IOROWS)],
                        out_hbm.at[c, pl.ds(io_base, IOROWS)])


_sc_gather_scatter = pl.kernel(
    _sc_body,
    out_type=jax.ShapeDtypeStruct((NC, N, D), jnp.float32),
    mesh=plsc.VectorSubcoreMesh(
        core_axis_name="c", subcore_axis_name="s", num_cores=NC, num_subcores=NS
    ),
    scratch_types=[
        pltpu.VMEM_SHARED((ACC_ROWS, D), jnp.float32),
        pltpu.VMEM((NCH_PH, K), jnp.int32),
        pltpu.VMEM((NCH_PH, K), jnp.int32),
        pltpu.VMEM((K, D), jnp.float32),
        pltpu.VMEM((K, D), jnp.float32),
        pltpu.SemaphoreType.DMA,
        pltpu.SemaphoreType.DMA,
    ],
)


def _combine_body(p_ref, f_ref, o_ref):
    o_ref[...] = p_ref[0] + p_ref[1] - f_ref[...]


_ROWS_BLK = 1000


def _combine(partials, feat):
    return pl.pallas_call(
        _combine_body,
        grid=(N // _ROWS_BLK,),
        in_specs=[
            pl.BlockSpec((NC, _ROWS_BLK, D), lambda i: (0, i, 0)),
            pl.BlockSpec((_ROWS_BLK, D), lambda i: (i, 0)),
        ],
        out_specs=pl.BlockSpec((_ROWS_BLK, D), lambda i: (i, 0)),
        out_shape=jax.ShapeDtypeStruct((N, D), jnp.float32),
    )(partials, feat)


@jax.jit
def kernel(feat, edge_index):
    ei = edge_index.astype(jnp.int32)
    npad = EPAD - E
    src = jnp.concatenate([ei[0], jnp.zeros((npad,), jnp.int32)])
    pad_dst = N + (jnp.arange(npad, dtype=jnp.int32) % JUNK)
    dst = jnp.concatenate([ei[1], pad_dst])
    src = src.reshape(NW * NCHUNK, K)
    dst = dst.reshape(NW * NCHUNK, K)
    partials = _sc_gather_scatter(feat, src, dst)
    return _combine(partials, feat)


# speedup vs baseline: 2.5992x; 2.5992x over previous
import jax
import jax.numpy as jnp
from jax import lax
from jax.experimental import pallas as pl
from jax.experimental.pallas import tpu as pltpu
from jax.experimental.pallas import tpu_sc as plsc

N = 10000
D = 128
DH = D // 2
E = 320000
NC = 2
NS = 16
K = 128
CT = 160
EPAD = NS * CT * K
NCH_PH = 40
NPH = CT // NCH_PH
JUNK = 512
ACC_ROWS = N + JUNK
IOSUB = 10
IOROWS = N // IOSUB


def _sc_body(feath_hbm, src_hbm, dst_hbm, out_hbm,
             feats, acc, idx_s, idx_d, rows0, rows1, sem0, sem1):
    c = lax.axis_index("c")
    s = lax.axis_index("s")

    io_base = pl.multiple_of(s * IOROWS, 8)

    @pl.when(s < IOSUB)
    def _init():
        pltpu.sync_copy(feath_hbm.at[c, pl.ds(io_base, IOROWS)],
                        feats.at[pl.ds(io_base, IOROWS)])
        pltpu.sync_copy(feath_hbm.at[c, pl.ds(io_base, IOROWS)],
                        acc.at[pl.ds(io_base, IOROWS)])

    plsc.subcore_barrier()

    for p in range(NPH):
        slab = pl.multiple_of(s * CT + p * NCH_PH, 8)
        pltpu.sync_copy(src_hbm.at[pl.ds(slab, NCH_PH)], idx_s)
        pltpu.sync_copy(dst_hbm.at[pl.ds(slab, NCH_PH)], idx_d)

        pltpu.async_copy(feats.at[idx_s.at[0]], rows0, sem0)

        def body(i, carry):
            j0 = 2 * i
            j1 = j0 + 1
            pltpu.async_copy(feats.at[idx_s.at[j1]], rows1, sem1)
            pltpu.make_async_copy(feats.at[idx_s.at[j0]], rows0, sem0).wait()
            pltpu.sync_copy(rows0, acc.at[idx_d.at[j0]], add=True)

            @pl.when(j1 + 1 < NCH_PH)
            def _():
                pltpu.async_copy(feats.at[idx_s.at[j1 + 1]], rows0, sem0)

            pltpu.make_async_copy(feats.at[idx_s.at[j1]], rows1, sem1).wait()
            pltpu.sync_copy(rows1, acc.at[idx_d.at[j1]], add=True)
            return carry

        lax.fori_loop(0, NCH_PH // 2, body, 0)

    plsc.subcore_barrier()

    @pl.when(s < IOSUB)
    def _writeout():
        pltpu.sync_copy(acc.at[pl.ds(io_base, IOROWS)],
                        out_hbm.at[c, pl.ds(io_base, IOROWS)])


_sc_gin = pl.kernel(
    _sc_body,
    out_type=jax.ShapeDtypeStruct((NC, N, DH), jnp.float32),
    mesh=plsc.VectorSubcoreMesh(
        core_axis_name="c", subcore_axis_name="s", num_cores=NC, num_subcores=NS
    ),
    scratch_types=[
        pltpu.VMEM_SHARED((N, DH), jnp.float32),
        pltpu.VMEM_SHARED((ACC_ROWS, DH), jnp.float32),
        pltpu.VMEM((NCH_PH, K), jnp.int32),
        pltpu.VMEM((NCH_PH, K), jnp.int32),
        pltpu.VMEM((K, DH), jnp.float32),
        pltpu.VMEM((K, DH), jnp.float32),
        pltpu.SemaphoreType.DMA,
        pltpu.SemaphoreType.DMA,
    ],
)


@jax.jit
def kernel(feat, edge_index):
    ei = edge_index.astype(jnp.int32)
    npad = EPAD - E
    src = jnp.concatenate([ei[0], jnp.zeros((npad,), jnp.int32)])
    pad_dst = N + (jnp.arange(npad, dtype=jnp.int32) % JUNK)
    dst = jnp.concatenate([ei[1], pad_dst])
    src = src.reshape(NS * CT, K)
    dst = dst.reshape(NS * CT, K)
    feat_halves = jnp.moveaxis(feat.reshape(N, NC, DH), 1, 0)
    out_halves = _sc_gin(feat_halves, src, dst)
    return jnp.moveaxis(out_halves, 0, 1).reshape(N, D)
